# baseline (speedup 1.0000x reference)
import functools

import jax
import jax.numpy as jnp
from jax import lax
from jax.experimental import pallas as pl
from jax.experimental.pallas import tpu as pltpu
from jax.experimental.pallas import tpu_sc as plsc

_NC, _NS = 2, 16
_NW = _NC * _NS
_SUP = 512

_SC_PARAMS = pltpu.CompilerParams(use_tc_tiling_on_sc=False,
                                  needs_layout_passes=False)
_MESH = dict(core_axis_name="c", subcore_axis_name="s")


def _pad_edges(src, et, dst, trash):
    e = src.shape[0]
    per = _NS * _SUP
    n_sup = -(-e // per)
    n_sup += n_sup % 2
    ep = n_sup * per
    pad = ep + _SUP - e
    srcp = jnp.concatenate([src, jnp.zeros((pad,), jnp.int32)])
    etp = jnp.concatenate([et, jnp.zeros((pad,), jnp.int32)])
    dstp = jnp.concatenate([dst, jnp.full((pad,), trash, jnp.int32)])
    return srcp, etp, dstp, dstp.reshape(-1, 128), ep


def _deg_hist(dstp, n_pad):
    e = dstp.shape[0]
    n_full = e // _SUP

    @functools.partial(
        pl.kernel, mesh=plsc.VectorSubcoreMesh(**_MESH),
        out_type=jax.ShapeDtypeStruct((_NW, n_pad), jnp.float32),
        scratch_types=[pltpu.VMEM((_SUP,), jnp.int32),
                       pltpu.VMEM((n_pad,), jnp.float32)],
        compiler_params=_SC_PARAMS,
    )
    def k(dst_hbm, out_hbm, dst_v, hist_v):
        wid = lax.axis_index("s") * _NC + lax.axis_index("c")
        ones = jnp.ones((16,), jnp.float32)

        def zero(i, c):
            hist_v[pl.ds(i * 16, 16)] = jnp.zeros((16,), jnp.float32)
            return c

        lax.fori_loop(0, n_pad // 16, zero, 0)

        def body(i, c):
            base = (wid + i * _NW) * _SUP
            pltpu.sync_copy(dst_hbm.at[pl.ds(base, _SUP)], dst_v)

            def vj(j, cc):
                plsc.addupdate_scatter(hist_v, [dst_v[pl.ds(j * 16, 16)]],
                                       ones)
                return cc

            lax.fori_loop(0, _SUP // 16, vj, 0)
            return c

        n_my = (n_full - wid + _NW - 1) // _NW
        lax.fori_loop(0, n_my, body, 0)
        pltpu.sync_copy(hist_v, out_hbm.at[wid])

    return k(dstp)


def _conv_agg(y_flat, srcp, etp, dst2d, n_nodes, n_pad, ep, w, t=7):
    per_tile = ep // _NS
    n_sup = per_tile // _SUP
    rpt = n_pad // _NS
    nv = _SUP // 16

    @functools.partial(
        pl.kernel, mesh=plsc.VectorSubcoreMesh(**_MESH),
        out_type=jax.ShapeDtypeStruct((_NC, n_pad, w), jnp.float32),
        scratch_types=[
            pltpu.VMEM((_SUP,), jnp.int32),
            pltpu.VMEM((_SUP,), jnp.int32),
            pltpu.VMEM((_SUP,), jnp.int32),
            pltpu.VMEM((_SUP,), jnp.int32),
            pltpu.VMEM((_SUP // 128, 128), jnp.int32),
            pltpu.VMEM((_SUP // 128, 128), jnp.int32),
            pltpu.VMEM((_SUP // 128, 128), jnp.int32),
            pltpu.VMEM((_SUP // 128, 128), jnp.int32),
            pltpu.VMEM((_SUP,), jnp.int32),
            pltpu.VMEM((_SUP,), jnp.int32),
            pltpu.VMEM((_SUP, w), jnp.float32),
            pltpu.VMEM((_SUP, w), jnp.float32),
            pltpu.VMEM((128, w), jnp.float32),
            pltpu.VMEM_SHARED((n_pad, w), jnp.float32),
            pltpu.SemaphoreType.DMA,
            pltpu.SemaphoreType.DMA,
            pltpu.SemaphoreType.DMA,
            pltpu.SemaphoreType.DMA,
            pltpu.SemaphoreType.DMA,
            pltpu.SemaphoreType.DMA,
        ],
        compiler_params=_SC_PARAMS,
    )
    def k(y_hbm, src_hbm, et_hbm, dst_hbm, out_hbm,
          srcb0, etb0, srcb1, etb1, dstb0, dstb1, dsts0, dsts1, idx0, idx1,
          rows0, rows1, zbuf, agg_sh, e0, e1, g0s, g1s, s0s, s1s):
        cid = lax.axis_index("c")
        sid = lax.axis_index("s")

        def zrow(r, c):
            for j in range(w // 16):
                zbuf[r, pl.ds(j * 16, 16)] = jnp.zeros((16,), jnp.float32)
            return c

        lax.fori_loop(0, 128, zrow, 0)

        def zcp(i, c):
            pltpu.sync_copy(zbuf, agg_sh.at[pl.ds(sid * rpt + i * 128, 128)])
            return c

        lax.fori_loop(0, rpt // 128, zcp, 0)
        if rpt % 128:
            pltpu.sync_copy(
                zbuf.at[pl.ds(0, rpt % 128)],
                agg_sh.at[pl.ds(sid * rpt + (rpt // 128) * 128, rpt % 128)])
        plsc.subcore_barrier()

        cbase = cid * (t * n_nodes)
        tbase = sid * per_tile

        def issue_edges(g, srcb, etb, dstb, sem):
            b = tbase + g * _SUP
            pltpu.async_copy(src_hbm.at[pl.ds(b, _SUP)], srcb, sem)
            pltpu.async_copy(et_hbm.at[pl.ds(b, _SUP)], etb, sem)
            pltpu.async_copy(dst_hbm.at[pl.ds(b // 128, _SUP // 128)], dstb, sem)

        def drain_edges(srcb, etb, dstb, sem):
            pltpu.make_async_copy(src_hbm.at[pl.ds(0, _SUP)], srcb,
                                  sem).wait()
            pltpu.make_async_copy(et_hbm.at[pl.ds(0, _SUP)], etb, sem).wait()
            pltpu.make_async_copy(dst_hbm.at[pl.ds(0, _SUP // 128)], dstb,
                                  sem).wait()

        def compute_idx(srcb, etb, idxb, dstb, dsts):
            def vj(j, c):
                sl = pl.ds(j * 16, 16)
                idxb[sl] = etb[sl] * n_nodes + srcb[sl] + cbase
                co = (j % 8) * 16
                dsts[j // 8, pl.ds(co, 16)] = dstb[j // 8, pl.ds(co, 16)]
                return c

            lax.fori_loop(0, nv, vj, 0)

        def fire_gathers(idxb, rowsb, sem):
            for j in range(_SUP // 128):
                pltpu.async_copy(y_hbm.at[idxb.at[pl.ds(j * 128, 128)]],
                                 rowsb.at[pl.ds(j * 128, 128)], sem)

        def fire_scatters(rowsb, dstb, sem):
            for j in range(_SUP // 128):
                pltpu.async_copy(rowsb.at[pl.ds(j * 128, 128)],
                                 agg_sh.at[dstb.at[j]], sem, add=True)

        def drain_rows(rowsb, sem):
            pltpu.make_async_copy(y_hbm.at[pl.ds(0, _SUP)], rowsb,
                                  sem).wait()

        issue_edges(0, srcb0, etb0, dstb0, e0)

        def body(kk, c):
            g0 = 2 * kk
            drain_edges(srcb0, etb0, dstb0, e0)
            issue_edges(g0 + 1, srcb1, etb1, dstb1, e1)

            @pl.when(kk > 0)
            def _():
                drain_rows(rows0, s0s)

            compute_idx(srcb0, etb0, idx0, dstb0, dsts0)
            fire_gathers(idx0, rows0, g0s)
            drain_edges(srcb1, etb1, dstb1, e1)
            issue_edges(g0 + 2, srcb0, etb0, dstb0, e0)

            @pl.when(kk > 0)
            def _():
                drain_rows(rows1, s1s)

            compute_idx(srcb1, etb1, idx1, dstb1, dsts1)
            fire_gathers(idx1, rows1, g1s)
            drain_rows(rows0, g0s)
            fire_scatters(rows0, dsts0, s0s)
            drain_rows(rows1, g1s)
            fire_scatters(rows1, dsts1, s1s)
            return c

        lax.fori_loop(0, n_sup // 2, body, 0)
        drain_edges(srcb0, etb0, dstb0, e0)
        drain_rows(rows0, s0s)
        drain_rows(rows1, s1s)
        plsc.subcore_barrier()

        def wb(i, c):
            off = sid * rpt + i * 128
            pltpu.sync_copy(agg_sh.at[pl.ds(off, 128)],
                            out_hbm.at[cid].at[pl.ds(off, 128)])
            return c

        lax.fori_loop(0, rpt // 128, wb, 0)
        if rpt % 128:
            off = sid * rpt + (rpt // 128) * 128
            pltpu.sync_copy(agg_sh.at[pl.ds(off, rpt % 128)],
                            out_hbm.at[cid].at[pl.ds(off, rpt % 128)])

    return k(y_flat, srcp, etp, dst2d)


def _upsample_gather(table, idx):
    n_out = idx.shape[0]
    c = table.shape[1]
    ch = 128
    n_full = n_out // ch
    tail = n_out - n_full * ch

    @functools.partial(
        pl.kernel, mesh=plsc.VectorSubcoreMesh(**_MESH),
        out_type=jax.ShapeDtypeStruct((n_out, c), jnp.float32),
        scratch_types=[pltpu.VMEM((ch,), jnp.int32),
                       pltpu.VMEM((ch, c), jnp.float32),
                       pltpu.SemaphoreType.DMA],
        compiler_params=_SC_PARAMS,
    )
    def k(table_hbm, idx_hbm, out_hbm, idx_v, rows_v, sem):
        wid = lax.axis_index("s") * _NC + lax.axis_index("c")
        n_my = (n_full - wid + _NW - 1) // _NW

        def body(i, carry):
            base = (wid + i * _NW) * ch
            pltpu.sync_copy(idx_hbm.at[pl.ds(base, ch)], idx_v)
            pltpu.async_copy(table_hbm.at[idx_v], rows_v, sem).wait()
            pltpu.sync_copy(rows_v, out_hbm.at[pl.ds(base, ch)])
            return carry

        lax.fori_loop(0, n_my, body, 0)

        if tail:
            @pl.when(wid == 1)
            def _():
                base = n_full * ch
                pltpu.sync_copy(idx_hbm.at[pl.ds(base, tail)],
                                idx_v.at[pl.ds(0, tail)])
                pltpu.async_copy(table_hbm.at[idx_v.at[pl.ds(0, tail)]],
                                 rows_v.at[pl.ds(0, tail)], sem).wait()
                pltpu.sync_copy(rows_v.at[pl.ds(0, tail)],
                                out_hbm.at[pl.ds(base, tail)])

    return k(table, idx)


def _gn_act(xb, ag, g, b):
    mean = jnp.dot(xb, ag, preferred_element_type=jnp.float32)
    var = jnp.dot(xb * xb, ag, preferred_element_type=jnp.float32) - mean * mean
    xn = (xb - mean) * lax.rsqrt(var + 1e-5)
    return jax.nn.gelu(xn * g + b)


def _split_cols(yb, o_ref, t, half):
    for k in range(2 * t):
        o_ref[k] = yb[:, k * half:(k + 1) * half]


def _agg_x(agg_ref, degp_ref):
    xb = jnp.concatenate([agg_ref[0], agg_ref[1]], axis=1)
    deg = jnp.sum(degp_ref[...], axis=1, keepdims=True)
    return xb * (1.0 / jnp.maximum(deg, 1.0))


def _pre_body(x_ref, ag_ref, g_ref, b_ref, w_ref, o_ref):
    act = _gn_act(x_ref[...], ag_ref[...], g_ref[...], b_ref[...])
    o_ref[...] = jnp.dot(act, w_ref[0], preferred_element_type=jnp.float32)


def _mid_body(agg_ref, degp_ref, ag_ref, g_ref, b_ref, w_ref, o_ref):
    act = _gn_act(_agg_x(agg_ref, degp_ref), ag_ref[...], g_ref[...],
                  b_ref[...])
    o_ref[...] = jnp.dot(act, w_ref[0], preferred_element_type=jnp.float32)


def _up_body(agg_ref, degp_ref, r_ref, ag_ref, g_ref, b_ref, w_ref, bb_ref,
             o_ref):
    xb = r_ref[...] + _agg_x(agg_ref, degp_ref)
    act = _gn_act(xb, ag_ref[...], g_ref[...], b_ref[...])
    o_ref[...] = jnp.dot(act, w_ref[...],
                         preferred_element_type=jnp.float32) + bb_ref[...]


def _fin_body(agg_ref, degp_ref, r_ref, o_ref):
    o_ref[...] = r_ref[...] + _agg_x(agg_ref, degp_ref)


def _avg_mat(c, groups=8):
    gs = c // groups
    eye = jnp.eye(groups, dtype=jnp.float32) / gs
    return jnp.kron(eye, jnp.ones((gs, gs), jnp.float32))


def _wcat(W):
    t, c, co = W.shape
    half = co // 2
    return jnp.transpose(W.reshape(t, c, 2, half),
                         (1, 2, 0, 3)).reshape(c, 2 * t * half)


def _row_spec(blk, c):
    return pl.BlockSpec((blk, c), lambda i: (i, 0))


def _full_spec(shape):
    nd = len(shape)
    return pl.BlockSpec(shape, lambda *g, _n=nd: (0,) * _n)


def _tc_pre(x, g, b, W, blk):
    n, c = x.shape
    t = W.shape[0]
    half = W.shape[2] // 2
    nb = n // blk
    return pl.pallas_call(
        _pre_body,
        grid=(nb, 2 * t),
        in_specs=[pl.BlockSpec((blk, c), lambda i, k: (i, 0)),
                  _full_spec((c, c)), _full_spec((1, c)), _full_spec((1, c)),
                  pl.BlockSpec((1, c, half), lambda i, k: (k, 0, 0))],
        out_specs=pl.BlockSpec((blk, half),
                               lambda i, k, _nb=nb: (k * _nb + i, 0)),
        out_shape=jax.ShapeDtypeStruct((2 * t * n, half), jnp.float32),
    )(x, _avg_mat(c), g.reshape(1, c), b.reshape(1, c),
      _wcat(W).T.reshape(2 * t, half, c).transpose(0, 2, 1))


def _agg_specs(blk, half):
    return [pl.BlockSpec((2, blk, half), lambda i: (0, i, 0)),
            pl.BlockSpec((blk, _NW), lambda i: (i, 0))]


def _tc_mid(agg2, degpt, g, b, W, blk):
    t, c = W.shape[0], W.shape[1]
    half = c // 2
    nn = (agg2.shape[1] // blk) * blk
    nb = nn // blk
    return pl.pallas_call(
        _mid_body,
        grid=(nb, 2 * t),
        in_specs=[pl.BlockSpec((2, blk, half), lambda i, k: (0, i, 0)),
                  pl.BlockSpec((blk, _NW), lambda i, k: (i, 0)),
                  _full_spec((c, c)), _full_spec((1, c)), _full_spec((1, c)),
                  pl.BlockSpec((1, c, half), lambda i, k: (k, 0, 0))],
        out_specs=pl.BlockSpec((blk, half),
                               lambda i, k, _nb=nb: (k * _nb + i, 0)),
        out_shape=jax.ShapeDtypeStruct((2 * t * nn, half), jnp.float32),
    )(agg2, degpt, _avg_mat(c), g.reshape(1, c), b.reshape(1, c),
      _wcat(W).T.reshape(2 * t, half, c).transpose(0, 2, 1))


def _tc_up(agg2, degpt, resid, g, b, W_up, b_upb, blk):
    n, c = resid.shape
    co = W_up.shape[1]
    half = c // 2
    return pl.pallas_call(
        _up_body,
        grid=(n // blk,),
        in_specs=_agg_specs(blk, half) + [
            _row_spec(blk, c), _full_spec((c, c)), _full_spec((1, c)),
            _full_spec((1, c)), _full_spec((c, co)), _full_spec((1, co))],
        out_specs=_row_spec(blk, co),
        out_shape=jax.ShapeDtypeStruct((n, co), jnp.float32),
    )(agg2, degpt, resid, _avg_mat(c), g.reshape(1, c), b.reshape(1, c),
      W_up, b_upb.reshape(1, co))


def _tc_fin(agg2, degpt, resid, blk):
    n, c = resid.shape
    half = c // 2
    return pl.pallas_call(
        _fin_body,
        grid=(n // blk,),
        in_specs=_agg_specs(blk, half) + [_row_spec(blk, c)],
        out_specs=_row_spec(blk, c),
        out_shape=jax.ShapeDtypeStruct((n, c), jnp.float32),
    )(agg2, degpt, resid)


def _add_kernel(a_ref, b_ref, o_ref):
    o_ref[...] = a_ref[...] + b_ref[...]


def _pallas_add(a, b):
    N, C = a.shape
    blk = 2000
    return pl.pallas_call(
        _add_kernel,
        grid=(N // blk,),
        in_specs=[pl.BlockSpec((blk, C), lambda i: (i, 0)),
                  pl.BlockSpec((blk, C), lambda i: (i, 0))],
        out_specs=pl.BlockSpec((blk, C), lambda i: (i, 0)),
        out_shape=jax.ShapeDtypeStruct((N, C), a.dtype),
    )(a, b)


def kernel(datas0, datas1, edge_index0, edge_type0, edge_index1, edge_type1,
           parent_idx, g0a, b0a, W1_0, g0b, b0b, W2_0, g_up, b_up, W_up,
           b_upb, g1a, b1a, W1_1, g1b, b1b, W2_1):
    n0, n1 = datas0.shape[0], datas1.shape[0]
    n0p = ((n0 + 127) // 128) * 128
    n1p = ((n1 + 127) // 128) * 128
    b0, b1 = 3136, 2000
    datas0p = jnp.concatenate(
        [datas0, jnp.zeros((n0p - n0, datas0.shape[1]), jnp.float32)])

    src0p, et0p, dst0p, dst0_2d, ep0 = _pad_edges(
        edge_index0[0], edge_type0, edge_index0[1], n0p - 1)
    src1p, et1p, dst1p, dst1_2d, ep1 = _pad_edges(
        edge_index1[0], edge_type1, edge_index1[1], n1p - 1)

    degpt0 = _deg_hist(dst0p, n0p).T
    degpt1 = _deg_hist(dst1p, n1p).T

    y = _tc_pre(datas0p, g0a, b0a, W1_0, b0)
    agg = _conv_agg(y, src0p, et0p, dst0_2d, n0p, n0p, ep0, 32)
    y = _tc_mid(agg, degpt0, g0b, b0b, W2_0, b0)
    agg = _conv_agg(y, src0p, et0p, dst0_2d, n0p, n0p, ep0, 32)

    h = _tc_up(agg, degpt0, datas0p, g_up, b_up, W_up, b_upb, b0)
    x1 = _pallas_add(_upsample_gather(h, parent_idx), datas1)

    y = _tc_pre(x1, g1a, b1a, W1_1, b1)
    agg = _conv_agg(y, src1p, et1p, dst1_2d, n1, n1p, ep1, 16)
    y = _tc_mid(agg, degpt1, g1b, b1b, W2_1, b1)
    agg = _conv_agg(y, src1p, et1p, dst1_2d, n1, n1p, ep1, 16)
    return _tc_fin(agg, degpt1, x1, b1)


# speedup vs baseline: 1.9430x; 1.9430x over previous
import functools

import jax
import jax.numpy as jnp
from jax import lax
from jax.experimental import pallas as pl
from jax.experimental.pallas import tpu as pltpu
from jax.experimental.pallas import tpu_sc as plsc

_NC, _NS = 2, 16
_NW = _NC * _NS
_SUP = 512

_SC_PARAMS = pltpu.CompilerParams(use_tc_tiling_on_sc=False,
                                  needs_layout_passes=False)
_MESH = dict(core_axis_name="c", subcore_axis_name="s")


def _pad_edges(src, et, dst, trash):
    e = src.shape[0]
    per = _NS * _SUP
    n_sup = -(-e // per)
    n_sup += n_sup % 2
    ep = n_sup * per
    pad = ep + _SUP - e
    srcp = jnp.concatenate([src, jnp.zeros((pad,), jnp.int32)])
    etp = jnp.concatenate([et, jnp.zeros((pad,), jnp.int32)])
    dstp = jnp.concatenate([dst, jnp.full((pad,), trash, jnp.int32)])
    return srcp, etp, dstp, dstp.reshape(-1, 128), ep


def _deg_hist(dstp, n_pad):
    e = dstp.shape[0]
    n_full = e // _SUP

    @functools.partial(
        pl.kernel, mesh=plsc.VectorSubcoreMesh(**_MESH),
        out_type=jax.ShapeDtypeStruct((_NW, n_pad), jnp.float32),
        scratch_types=[pltpu.VMEM((_SUP,), jnp.int32),
                       pltpu.VMEM((n_pad,), jnp.float32)],
        compiler_params=_SC_PARAMS,
    )
    def k(dst_hbm, out_hbm, dst_v, hist_v):
        wid = lax.axis_index("s") * _NC + lax.axis_index("c")
        ones = jnp.ones((16,), jnp.float32)

        def zero(i, c):
            hist_v[pl.ds(i * 16, 16)] = jnp.zeros((16,), jnp.float32)
            return c

        lax.fori_loop(0, n_pad // 16, zero, 0)

        def body(i, c):
            base = (wid + i * _NW) * _SUP
            pltpu.sync_copy(dst_hbm.at[pl.ds(base, _SUP)], dst_v)

            def vj(j, cc):
                plsc.addupdate_scatter(hist_v, [dst_v[pl.ds(j * 16, 16)]],
                                       ones)
                return cc

            lax.fori_loop(0, _SUP // 16, vj, 0)
            return c

        n_my = (n_full - wid + _NW - 1) // _NW
        lax.fori_loop(0, n_my, body, 0)
        pltpu.sync_copy(hist_v, out_hbm.at[wid])

    return k(dstp)


def _conv_agg(y_flat, srcp, etp, dst2d, n_nodes, n_pad, ep, w, t=7):
    per_tile = ep // _NS
    n_sup = per_tile // _SUP
    rpt = n_pad // _NS
    nv = _SUP // 16

    @functools.partial(
        pl.kernel, mesh=plsc.VectorSubcoreMesh(**_MESH),
        out_type=jax.ShapeDtypeStruct((_NC, n_pad, w), jnp.float32),
        scratch_types=[
            pltpu.VMEM((_SUP,), jnp.int32),
            pltpu.VMEM((_SUP,), jnp.int32),
            pltpu.VMEM((_SUP,), jnp.int32),
            pltpu.VMEM((_SUP,), jnp.int32),
            pltpu.VMEM((_SUP // 128, 128), jnp.int32),
            pltpu.VMEM((_SUP // 128, 128), jnp.int32),
            pltpu.VMEM((_SUP // 128, 128), jnp.int32),
            pltpu.VMEM((_SUP // 128, 128), jnp.int32),
            pltpu.VMEM((_SUP,), jnp.int32),
            pltpu.VMEM((_SUP,), jnp.int32),
            pltpu.VMEM((_SUP, w), jnp.float32),
            pltpu.VMEM((_SUP, w), jnp.float32),
            pltpu.VMEM((128, w), jnp.float32),
            pltpu.VMEM_SHARED((n_pad, w), jnp.float32),
            pltpu.SemaphoreType.DMA,
            pltpu.SemaphoreType.DMA,
            pltpu.SemaphoreType.DMA,
            pltpu.SemaphoreType.DMA,
            pltpu.SemaphoreType.DMA,
            pltpu.SemaphoreType.DMA,
        ],
        compiler_params=_SC_PARAMS,
    )
    def k(y_hbm, src_hbm, et_hbm, dst_hbm, out_hbm,
          srcb0, etb0, srcb1, etb1, dstb0, dstb1, dsts0, dsts1, idx0, idx1,
          rows0, rows1, zbuf, agg_sh, e0, e1, g0s, g1s, s0s, s1s):
        cid = lax.axis_index("c")
        sid = lax.axis_index("s")

        def zrow(r, c):
            for j in range(w // 16):
                zbuf[r, pl.ds(j * 16, 16)] = jnp.zeros((16,), jnp.float32)
            return c

        lax.fori_loop(0, 128, zrow, 0)

        def zcp(i, c):
            pltpu.sync_copy(zbuf, agg_sh.at[pl.ds(sid * rpt + i * 128, 128)])
            return c

        lax.fori_loop(0, rpt // 128, zcp, 0)
        if rpt % 128:
            pltpu.sync_copy(
                zbuf.at[pl.ds(0, rpt % 128)],
                agg_sh.at[pl.ds(sid * rpt + (rpt // 128) * 128, rpt % 128)])
        plsc.subcore_barrier()

        cbase = cid * (t * n_nodes)
        tbase = sid * per_tile

        def issue_edges(g, srcb, etb, dstb, sem):
            b = tbase + g * _SUP
            pltpu.async_copy(src_hbm.at[pl.ds(b, _SUP)], srcb, sem)
            pltpu.async_copy(et_hbm.at[pl.ds(b, _SUP)], etb, sem)
            pltpu.async_copy(dst_hbm.at[pl.ds(b // 128, _SUP // 128)], dstb, sem)

        def drain_edges(srcb, etb, dstb, sem):
            pltpu.make_async_copy(src_hbm.at[pl.ds(0, _SUP)], srcb,
                                  sem).wait()
            pltpu.make_async_copy(et_hbm.at[pl.ds(0, _SUP)], etb, sem).wait()
            pltpu.make_async_copy(dst_hbm.at[pl.ds(0, _SUP // 128)], dstb,
                                  sem).wait()

        def compute_idx(srcb, etb, idxb, dstb, dsts):
            def vj(j, c):
                sl = pl.ds(j * 16, 16)
                idxb[sl] = etb[sl] * n_nodes + srcb[sl] + cbase
                co = (j % 8) * 16
                dsts[j // 8, pl.ds(co, 16)] = dstb[j // 8, pl.ds(co, 16)]
                return c

            lax.fori_loop(0, nv, vj, 0)

        def fire_gathers(idxb, rowsb, sem):
            for j in range(_SUP // 128):
                pltpu.async_copy(y_hbm.at[idxb.at[pl.ds(j * 128, 128)]],
                                 rowsb.at[pl.ds(j * 128, 128)], sem)

        def fire_scatters(rowsb, dstb, sem):
            for j in range(_SUP // 128):
                pltpu.async_copy(rowsb.at[pl.ds(j * 128, 128)],
                                 agg_sh.at[dstb.at[j]], sem, add=True)

        def drain_rows(rowsb, sem):
            pltpu.make_async_copy(y_hbm.at[pl.ds(0, _SUP)], rowsb,
                                  sem).wait()

        issue_edges(0, srcb0, etb0, dstb0, e0)

        def body(kk, c):
            g0 = 2 * kk
            drain_edges(srcb0, etb0, dstb0, e0)
            issue_edges(g0 + 1, srcb1, etb1, dstb1, e1)

            @pl.when(kk > 0)
            def _():
                drain_rows(rows0, s0s)

            compute_idx(srcb0, etb0, idx0, dstb0, dsts0)
            fire_gathers(idx0, rows0, g0s)
            drain_edges(srcb1, etb1, dstb1, e1)
            issue_edges(g0 + 2, srcb0, etb0, dstb0, e0)

            @pl.when(kk > 0)
            def _():
                drain_rows(rows1, s1s)

            compute_idx(srcb1, etb1, idx1, dstb1, dsts1)
            fire_gathers(idx1, rows1, g1s)
            drain_rows(rows0, g0s)
            fire_scatters(rows0, dsts0, s0s)
            drain_rows(rows1, g1s)
            fire_scatters(rows1, dsts1, s1s)
            return c

        lax.fori_loop(0, n_sup // 2, body, 0)
        drain_edges(srcb0, etb0, dstb0, e0)
        drain_rows(rows0, s0s)
        drain_rows(rows1, s1s)
        plsc.subcore_barrier()

        def wb(i, c):
            off = sid * rpt + i * 128
            pltpu.sync_copy(agg_sh.at[pl.ds(off, 128)],
                            out_hbm.at[cid].at[pl.ds(off, 128)])
            return c

        lax.fori_loop(0, rpt // 128, wb, 0)
        if rpt % 128:
            off = sid * rpt + (rpt // 128) * 128
            pltpu.sync_copy(agg_sh.at[pl.ds(off, rpt % 128)],
                            out_hbm.at[cid].at[pl.ds(off, rpt % 128)])

    return k(y_flat, srcp, etp, dst2d)


def _upsample_gather(table, idx):
    n_out = idx.shape[0]
    c = table.shape[1]
    ch = 128
    n_full = n_out // ch
    tail = n_out - n_full * ch

    @functools.partial(
        pl.kernel, mesh=plsc.VectorSubcoreMesh(**_MESH),
        out_type=jax.ShapeDtypeStruct((n_out, c), jnp.float32),
        scratch_types=[pltpu.VMEM((ch,), jnp.int32),
                       pltpu.VMEM((ch, c), jnp.float32),
                       pltpu.SemaphoreType.DMA],
        compiler_params=_SC_PARAMS,
    )
    def k(table_hbm, idx_hbm, out_hbm, idx_v, rows_v, sem):
        wid = lax.axis_index("s") * _NC + lax.axis_index("c")
        n_my = (n_full - wid + _NW - 1) // _NW

        def body(i, carry):
            base = (wid + i * _NW) * ch
            pltpu.sync_copy(idx_hbm.at[pl.ds(base, ch)], idx_v)
            pltpu.async_copy(table_hbm.at[idx_v], rows_v, sem).wait()
            pltpu.sync_copy(rows_v, out_hbm.at[pl.ds(base, ch)])
            return carry

        lax.fori_loop(0, n_my, body, 0)

        if tail:
            @pl.when(wid == 1)
            def _():
                base = n_full * ch
                pltpu.sync_copy(idx_hbm.at[pl.ds(base, tail)],
                                idx_v.at[pl.ds(0, tail)])
                pltpu.async_copy(table_hbm.at[idx_v.at[pl.ds(0, tail)]],
                                 rows_v.at[pl.ds(0, tail)], sem).wait()
                pltpu.sync_copy(rows_v.at[pl.ds(0, tail)],
                                out_hbm.at[pl.ds(base, tail)])

    return k(table, idx)


def _gn_act(xb, ag, g, b):
    mean = jnp.dot(xb, ag, preferred_element_type=jnp.float32)
    var = jnp.dot(xb * xb, ag, preferred_element_type=jnp.float32) - mean * mean
    xn = (xb - mean) * lax.rsqrt(var + 1e-5)
    return jax.nn.gelu(xn * g + b)


def _split_cols(yb, o_ref, t, half):
    for k in range(2 * t):
        o_ref[k] = yb[:, k * half:(k + 1) * half]


def _agg_x(agg_ref, degp_ref):
    xb = jnp.concatenate([agg_ref[0], agg_ref[1]], axis=1)
    deg = jnp.sum(degp_ref[...], axis=1, keepdims=True)
    return xb * (1.0 / jnp.maximum(deg, 1.0))


def _pre_body(x_ref, ag_ref, g_ref, b_ref, w_ref, o_ref, *, t, half):
    act = _gn_act(x_ref[...], ag_ref[...], g_ref[...], b_ref[...])
    yb = jnp.dot(act, w_ref[...], preferred_element_type=jnp.float32)
    _split_cols(yb, o_ref, t, half)


def _mid_body(agg_ref, degp_ref, ag_ref, g_ref, b_ref, w_ref, o_ref, *, t,
              half):
    act = _gn_act(_agg_x(agg_ref, degp_ref), ag_ref[...], g_ref[...],
                  b_ref[...])
    yb = jnp.dot(act, w_ref[...], preferred_element_type=jnp.float32)
    _split_cols(yb, o_ref, t, half)


def _up_body(agg_ref, degp_ref, r_ref, ag_ref, g_ref, b_ref, w_ref, bb_ref,
             o_ref):
    xb = r_ref[...] + _agg_x(agg_ref, degp_ref)
    act = _gn_act(xb, ag_ref[...], g_ref[...], b_ref[...])
    o_ref[...] = jnp.dot(act, w_ref[...],
                         preferred_element_type=jnp.float32) + bb_ref[...]


def _fin_body(agg_ref, degp_ref, r_ref, o_ref):
    o_ref[...] = r_ref[...] + _agg_x(agg_ref, degp_ref)


def _avg_mat(c, groups=8):
    gs = c // groups
    eye = jnp.eye(groups, dtype=jnp.float32) / gs
    return jnp.kron(eye, jnp.ones((gs, gs), jnp.float32))


def _wcat(W):
    t, c, co = W.shape
    half = co // 2
    return jnp.transpose(W.reshape(t, c, 2, half),
                         (1, 2, 0, 3)).reshape(c, 2 * t * half)


def _row_spec(blk, c):
    return pl.BlockSpec((blk, c), lambda i: (i, 0))


def _full_spec(shape):
    nd = len(shape)
    return pl.BlockSpec(shape, lambda *g, _n=nd: (0,) * _n)


def _tc_pre(x, g, b, W, blk):
    n, c = x.shape
    t = W.shape[0]
    half = W.shape[2] // 2
    return pl.pallas_call(
        functools.partial(_pre_body, t=t, half=half),
        grid=(n // blk,),
        in_specs=[_row_spec(blk, c), _full_spec((c, c)),
                  _full_spec((1, c)), _full_spec((1, c)),
                  _full_spec((c, 2 * t * half))],
        out_specs=pl.BlockSpec((2 * t, blk, half), lambda i: (0, i, 0)),
        out_shape=jax.ShapeDtypeStruct((2 * t, n, half), jnp.float32),
    )(x, _avg_mat(c), g.reshape(1, c), b.reshape(1, c), _wcat(W))


def _agg_specs(blk, half):
    return [pl.BlockSpec((2, blk, half), lambda i: (0, i, 0)),
            pl.BlockSpec((blk, _NW), lambda i: (i, 0))]


def _tc_mid(agg2, degpt, g, b, W, blk):
    t, c = W.shape[0], W.shape[1]
    half = c // 2
    nn = (agg2.shape[1] // blk) * blk
    return pl.pallas_call(
        functools.partial(_mid_body, t=t, half=half),
        grid=(nn // blk,),
        in_specs=_agg_specs(blk, half) + [
            _full_spec((c, c)), _full_spec((1, c)), _full_spec((1, c)),
            _full_spec((c, 2 * t * half))],
        out_specs=pl.BlockSpec((2 * t, blk, half), lambda i: (0, i, 0)),
        out_shape=jax.ShapeDtypeStruct((2 * t, nn, half), jnp.float32),
    )(agg2, degpt, _avg_mat(c), g.reshape(1, c), b.reshape(1, c), _wcat(W))


def _tc_up(agg2, degpt, resid, g, b, W_up, b_upb, blk):
    n, c = resid.shape
    co = W_up.shape[1]
    half = c // 2
    return pl.pallas_call(
        _up_body,
        grid=(n // blk,),
        in_specs=_agg_specs(blk, half) + [
            _row_spec(blk, c), _full_spec((c, c)), _full_spec((1, c)),
            _full_spec((1, c)), _full_spec((c, co)), _full_spec((1, co))],
        out_specs=_row_spec(blk, co),
        out_shape=jax.ShapeDtypeStruct((n, co), jnp.float32),
    )(agg2, degpt, resid, _avg_mat(c), g.reshape(1, c), b.reshape(1, c),
      W_up, b_upb.reshape(1, co))


def _tc_fin(agg2, degpt, resid, blk):
    n, c = resid.shape
    half = c // 2
    return pl.pallas_call(
        _fin_body,
        grid=(n // blk,),
        in_specs=_agg_specs(blk, half) + [_row_spec(blk, c)],
        out_specs=_row_spec(blk, c),
        out_shape=jax.ShapeDtypeStruct((n, c), jnp.float32),
    )(agg2, degpt, resid)


def _add_kernel(a_ref, b_ref, o_ref):
    o_ref[...] = a_ref[...] + b_ref[...]


def _pallas_add(a, b):
    N, C = a.shape
    blk = 2000
    return pl.pallas_call(
        _add_kernel,
        grid=(N // blk,),
        in_specs=[pl.BlockSpec((blk, C), lambda i: (i, 0)),
                  pl.BlockSpec((blk, C), lambda i: (i, 0))],
        out_specs=pl.BlockSpec((blk, C), lambda i: (i, 0)),
        out_shape=jax.ShapeDtypeStruct((N, C), a.dtype),
    )(a, b)


def kernel(datas0, datas1, edge_index0, edge_type0, edge_index1, edge_type1,
           parent_idx, g0a, b0a, W1_0, g0b, b0b, W2_0, g_up, b_up, W_up,
           b_upb, g1a, b1a, W1_1, g1b, b1b, W2_1):
    n0, n1 = datas0.shape[0], datas1.shape[0]
    n0p = ((n0 + 127) // 128) * 128
    n1p = ((n1 + 127) // 128) * 128
    b0, b1 = 3136, 2000
    datas0p = jnp.concatenate(
        [datas0, jnp.zeros((n0p - n0, datas0.shape[1]), jnp.float32)])

    src0p, et0p, dst0p, dst0_2d, ep0 = _pad_edges(
        edge_index0[0], edge_type0, edge_index0[1], n0p - 1)
    src1p, et1p, dst1p, dst1_2d, ep1 = _pad_edges(
        edge_index1[0], edge_type1, edge_index1[1], n1p - 1)

    degpt0 = _deg_hist(dst0p, n0p).T
    degpt1 = _deg_hist(dst1p, n1p).T

    y = _tc_pre(datas0p, g0a, b0a, W1_0, b0)
    agg = _conv_agg(y.reshape(-1, 32), src0p, et0p, dst0_2d, n0p, n0p, ep0, 32)
    y = _tc_mid(agg, degpt0, g0b, b0b, W2_0, b0)
    agg = _conv_agg(y.reshape(-1, 32), src0p, et0p, dst0_2d, n0p, n0p, ep0, 32)

    h = _tc_up(agg, degpt0, datas0p, g_up, b_up, W_up, b_upb, b0)
    x1 = _pallas_add(_upsample_gather(h, parent_idx), datas1)

    y = _tc_pre(x1, g1a, b1a, W1_1, b1)
    agg = _conv_agg(y.reshape(-1, 16), src1p, et1p, dst1_2d, n1, n1p, ep1, 16)
    y = _tc_mid(agg, degpt1, g1b, b1b, W2_1, b1)
    agg = _conv_agg(y.reshape(-1, 16), src1p, et1p, dst1_2d, n1, n1p, ep1, 16)
    return _tc_fin(agg, degpt1, x1, b1)
